# Initial kernel scaffold; baseline (speedup 1.0000x reference)
import jax
import jax.numpy as jnp
from jax.experimental import pallas as pl


def kernel(x, edge_index, W1, b1, W2, b2, Wout, bout):
    raise NotImplementedError("write your pallas kernel here")


# speedup vs baseline: 26.2926x; 26.2926x over previous
import functools

import jax
import jax.numpy as jnp
from jax import lax
from jax.experimental import pallas as pl
from jax.experimental.pallas import tpu as pltpu
from jax.experimental.pallas import tpu_sc as plsc

N = 10000
E = 320000
D = 128
H1 = 32
H2 = 16

NC = 2
NS = 16
NW = NC * NS
K = 128
C = -(-E // (NW * K))
EPW = C * K
EPAD = NW * EPW
RPT = 632
NP = NS * RPT
NA = NP

_MESH = dict(core_axis_name="c", subcore_axis_name="s")


def _make_agg(H):
  @functools.partial(
      pl.kernel,
      out_type=jax.ShapeDtypeStruct((NC, NP, H), jnp.float32),
      mesh=plsc.VectorSubcoreMesh(**_MESH),
      scratch_types=[
          pltpu.VMEM((C, K), jnp.int32),
          pltpu.VMEM((C, K), jnp.int32),
          pltpu.VMEM((K, H), jnp.float32),
          pltpu.VMEM_SHARED((NA, H), jnp.float32),
      ],
      compiler_params=pltpu.CompilerParams(use_tc_tiling_on_sc=False),
  )
  def agg(y, srcw, dstw, out, sidx, didx, rows, acc):
    cid = lax.axis_index("c")
    sid = lax.axis_index("s")
    wid = sid * NC + cid
    pltpu.sync_copy(srcw.at[wid], sidx)
    pltpu.sync_copy(dstw.at[wid], didx)
    pltpu.sync_copy(y.at[pl.ds(sid * RPT, RPT)], acc.at[pl.ds(sid * RPT, RPT)])
    plsc.subcore_barrier()

    def body(j, carry):
      pltpu.sync_copy(y.at[sidx.at[j]], rows)
      pltpu.sync_copy(rows, acc.at[didx.at[j]], add=True)
      return carry

    lax.fori_loop(0, C, body, 0)
    plsc.subcore_barrier()
    pltpu.sync_copy(acc.at[pl.ds(sid * RPT, RPT)],
                    out.at[cid, pl.ds(sid * RPT, RPT)])

  return agg


@functools.partial(
    pl.kernel,
    out_type=jax.ShapeDtypeStruct((NC, NP, 16), jnp.float32),
    mesh=plsc.VectorSubcoreMesh(**_MESH),
    scratch_types=[
        pltpu.VMEM((C, K), jnp.int32),
        pltpu.VMEM((K, 16), jnp.float32),
        pltpu.VMEM_SHARED((NA, 16), jnp.float32),
    ],
    compiler_params=pltpu.CompilerParams(use_tc_tiling_on_sc=False),
)
def _deg_sc(zeros_nh, ones_kh, dstw, out, didx, ones_v, acc):
  cid = lax.axis_index("c")
  sid = lax.axis_index("s")
  wid = sid * NC + cid
  pltpu.sync_copy(dstw.at[wid], didx)
  pltpu.sync_copy(ones_kh, ones_v)
  pltpu.sync_copy(zeros_nh.at[pl.ds(sid * RPT, RPT)],
                  acc.at[pl.ds(sid * RPT, RPT)])
  plsc.subcore_barrier()

  def body(j, carry):
    pltpu.sync_copy(ones_v, acc.at[didx.at[j]], add=True)
    return carry

  lax.fori_loop(0, C, body, 0)
  plsc.subcore_barrier()
  pltpu.sync_copy(acc.at[pl.ds(sid * RPT, RPT)],
                  out.at[cid, pl.ds(sid * RPT, RPT)])


_agg32 = _make_agg(H1)
_agg16 = _make_agg(H2)


def _tc1_body(degp, x, w1, dis_ref, y_ref):
  deg = degp[0, :, 0:1] + degp[1, :, 0:1] + 1.0
  dis = 1.0 / jnp.sqrt(deg)
  dis_ref[...] = dis
  y_ref[...] = jnp.dot(x[...], w1[...],
                       preferred_element_type=jnp.float32) * dis


def _tc2_body(zp, y1, dis, b1, w2, y2_ref):
  z = zp[0] + zp[1] - y1[...]
  h = jnp.maximum(z * dis[...] + b1[...], 0.0)
  y2_ref[...] = jnp.dot(h, w2[...],
                        preferred_element_type=jnp.float32) * dis[...]


def _tc3_body(zp, y2, dis, b2, wout, bout, emb_ref, logit_ref):
  z = zp[0] + zp[1] - y2[...]
  emb = jnp.maximum(z * dis[...] + b2[...], 0.0)
  emb_ref[...] = emb
  logit_ref[...] = jnp.dot(emb, wout[...],
                           preferred_element_type=jnp.float32) + bout[0, 0]


def kernel(x, edge_index, W1, b1, W2, b2, Wout, bout):
  src = edge_index[0]
  dst = edge_index[1]
  pad = EPAD - E
  srcw = jnp.concatenate(
      [src, jnp.zeros((pad,), jnp.int32)]).reshape(NW, C, K)
  dstw = jnp.concatenate(
      [dst, jnp.full((pad,), N, jnp.int32)]).reshape(NW, C, K)
  zeros_nh = jnp.zeros((NP, 16), jnp.float32)
  ones_kh = jnp.ones((K, 16), jnp.float32)

  degp = _deg_sc(zeros_nh, ones_kh, dstw)

  dis, y1 = pl.pallas_call(
      _tc1_body,
      out_shape=[
          jax.ShapeDtypeStruct((N, 1), jnp.float32),
          jax.ShapeDtypeStruct((N, H1), jnp.float32),
      ],
  )(degp[:, :N], x, W1)

  zp1 = _agg32(jnp.pad(y1, ((0, NP - N), (0, 0))), srcw, dstw)

  y2 = pl.pallas_call(
      _tc2_body,
      out_shape=jax.ShapeDtypeStruct((N, H2), jnp.float32),
  )(zp1[:, :N], y1, dis, b1.reshape(1, H1), W2)

  zp2 = _agg16(jnp.pad(y2, ((0, NP - N), (0, 0))), srcw, dstw)

  embedding, logits = pl.pallas_call(
      _tc3_body,
      out_shape=[
          jax.ShapeDtypeStruct((N, H2), jnp.float32),
          jax.ShapeDtypeStruct((N, 1), jnp.float32),
      ],
  )(zp2[:, :N], y2, dis, b2.reshape(1, H2), Wout, bout.reshape(1, 1))

  return (logits.squeeze(-1), embedding)


# speedup vs baseline: 58.8576x; 2.2386x over previous
import functools

import jax
import jax.numpy as jnp
from jax import lax
from jax.experimental import pallas as pl
from jax.experimental.pallas import tpu as pltpu
from jax.experimental.pallas import tpu_sc as plsc

N = 10000
E = 320000
D = 128
H1 = 32
H2 = 16

NC = 2
NS = 16
NW = NC * NS
K = 128
CH = E // K
C1 = -(-CH // NW)
C2 = CH - (NW - 1) * C1
NBUF = 4
RPT = 632
NP = NS * RPT
NA = NP

_MESH = dict(core_axis_name="c", subcore_axis_name="s")
_SC_PARAMS = pltpu.CompilerParams(use_tc_tiling_on_sc=False)


def _make_agg(H):
  @functools.partial(
      pl.kernel,
      out_type=jax.ShapeDtypeStruct((NC, NP, H), jnp.float32),
      mesh=plsc.VectorSubcoreMesh(**_MESH),
      scratch_types=[
          pltpu.VMEM((C1, K), jnp.int32),
          pltpu.VMEM((C1, K), jnp.int32),
          pltpu.VMEM((NBUF, K, H), jnp.float32),
          pltpu.VMEM_SHARED((NA, H), jnp.float32),
          pltpu.VMEM_SHARED((NA, H), jnp.float32),
          pltpu.SemaphoreType.DMA,
          pltpu.SemaphoreType.DMA,
      ],
      compiler_params=_SC_PARAMS,
  )
  def agg(y, ei3, out, sidx, didx, rows, acc, ysp, gsem, ssem):
    cid = lax.axis_index("c")
    sid = lax.axis_index("s")
    wid = sid * NC + cid
    cbase = wid * C1
    is_last = wid == NW - 1
    n = jnp.where(is_last, C2, C1)

    @pl.when(jnp.logical_not(is_last))
    def _():
      pltpu.sync_copy(ei3.at[0, pl.ds(cbase, C1)], sidx)
      pltpu.sync_copy(ei3.at[1, pl.ds(cbase, C1)], didx)

    @pl.when(is_last)
    def _():
      pltpu.sync_copy(ei3.at[0, pl.ds(cbase, C2)], sidx.at[pl.ds(0, C2)])
      pltpu.sync_copy(ei3.at[1, pl.ds(cbase, C2)], didx.at[pl.ds(0, C2)])

    pltpu.sync_copy(y.at[pl.ds(sid * RPT, RPT)], acc.at[pl.ds(sid * RPT, RPT)])
    pltpu.sync_copy(y.at[pl.ds(sid * RPT, RPT)], ysp.at[pl.ds(sid * RPT, RPT)])
    plsc.subcore_barrier()

    pltpu.async_copy(ysp.at[sidx.at[0]], rows.at[0], gsem)
    pltpu.async_copy(ysp.at[sidx.at[1]], rows.at[1], gsem)

    def body(j, carry):
      b = lax.rem(j, NBUF)

      @pl.when(j >= 2)
      def _():
        pltpu.make_async_copy(
            rows.at[lax.rem(j + 2, NBUF)],
            acc.at[didx.at[j - 2]], ssem).wait()

      @pl.when(j + 2 < n)
      def _():
        pltpu.async_copy(
            ysp.at[sidx.at[j + 2]], rows.at[lax.rem(j + 2, NBUF)], gsem)

      pltpu.make_async_copy(ysp.at[sidx.at[j]], rows.at[b], gsem).wait()
      pltpu.async_copy(rows.at[b], acc.at[didx.at[j]], ssem, add=True)
      return carry

    lax.fori_loop(0, n, body, 0)
    pltpu.make_async_copy(rows.at[0], acc.at[didx.at[0]], ssem).wait()
    pltpu.make_async_copy(rows.at[0], acc.at[didx.at[0]], ssem).wait()
    plsc.subcore_barrier()
    pltpu.sync_copy(acc.at[pl.ds(sid * RPT, RPT)],
                    out.at[cid, pl.ds(sid * RPT, RPT)])

  return agg


@functools.partial(
    pl.kernel,
    out_type=jax.ShapeDtypeStruct((NC, NP, 16), jnp.float32),
    mesh=plsc.VectorSubcoreMesh(**_MESH),
    scratch_types=[
        pltpu.VMEM((C1, K), jnp.int32),
        pltpu.VMEM((K, 16), jnp.float32),
        pltpu.VMEM_SHARED((NA, 16), jnp.float32),
        pltpu.SemaphoreType.DMA,
    ],
    compiler_params=_SC_PARAMS,
)
def _deg_sc(zeros_nh, ones_kh, ei3, out, didx, ones_v, acc, ssem):
  cid = lax.axis_index("c")
  sid = lax.axis_index("s")
  wid = sid * NC + cid
  cbase = wid * C1
  is_last = wid == NW - 1
  n = jnp.where(is_last, C2, C1)

  @pl.when(jnp.logical_not(is_last))
  def _():
    pltpu.sync_copy(ei3.at[1, pl.ds(cbase, C1)], didx)

  @pl.when(is_last)
  def _():
    pltpu.sync_copy(ei3.at[1, pl.ds(cbase, C2)], didx.at[pl.ds(0, C2)])

  pltpu.sync_copy(ones_kh, ones_v)
  pltpu.sync_copy(zeros_nh.at[pl.ds(sid * RPT, RPT)],
                  acc.at[pl.ds(sid * RPT, RPT)])
  plsc.subcore_barrier()

  def body(j, carry):
    @pl.when(j >= 4)
    def _():
      pltpu.make_async_copy(ones_v, acc.at[didx.at[0]], ssem).wait()

    pltpu.async_copy(ones_v, acc.at[didx.at[j]], ssem, add=True)
    return carry

  lax.fori_loop(0, n, body, 0)
  for _ in range(4):
    pltpu.make_async_copy(ones_v, acc.at[didx.at[0]], ssem).wait()
  plsc.subcore_barrier()
  pltpu.sync_copy(acc.at[pl.ds(sid * RPT, RPT)],
                  out.at[cid, pl.ds(sid * RPT, RPT)])


_agg32 = _make_agg(H1)
_agg16 = _make_agg(H2)


def _tc1_body(degp, x, w1, dis_ref, y_ref):
  deg = degp[0, :N, 0:1] + degp[1, :N, 0:1] + 1.0
  dis = 1.0 / jnp.sqrt(deg)
  dis_ref[...] = dis
  y_ref[0:N, :] = jnp.dot(x[...], w1[...],
                          preferred_element_type=jnp.float32) * dis


def _tc2_body(zp, y1, dis, b1, w2, y2_ref):
  z = zp[0, :N] + zp[1, :N] - y1[0:N, :]
  h = jnp.maximum(z * dis[...] + b1[...], 0.0)
  y2_ref[0:N, :] = jnp.dot(h, w2[...],
                           preferred_element_type=jnp.float32) * dis[...]


def _tc3_body(zp, y2, dis, b2, wout, bout, emb_ref, logit_ref):
  z = zp[0, :N] + zp[1, :N] - y2[0:N, :]
  emb = jnp.maximum(z * dis[...] + b2[...], 0.0)
  emb_ref[...] = emb
  logit_ref[...] = jnp.dot(emb, wout[...],
                           preferred_element_type=jnp.float32) + bout[0, 0]


def kernel(x, edge_index, W1, b1, W2, b2, Wout, bout):
  ei3 = edge_index.reshape(2, CH, K)
  zeros_nh = jnp.zeros((NP, 16), jnp.float32)
  ones_kh = jnp.ones((K, 16), jnp.float32)

  degp = _deg_sc(zeros_nh, ones_kh, ei3)

  dis, y1 = pl.pallas_call(
      _tc1_body,
      out_shape=[
          jax.ShapeDtypeStruct((N, 1), jnp.float32),
          jax.ShapeDtypeStruct((NP, H1), jnp.float32),
      ],
  )(degp, x, W1)

  zp1 = _agg32(y1, ei3)

  y2 = pl.pallas_call(
      _tc2_body,
      out_shape=jax.ShapeDtypeStruct((NP, H2), jnp.float32),
  )(zp1, y1, dis, b1.reshape(1, H1), W2)

  zp2 = _agg16(y2, ei3)

  embedding, logits = pl.pallas_call(
      _tc3_body,
      out_shape=[
          jax.ShapeDtypeStruct((N, H2), jnp.float32),
          jax.ShapeDtypeStruct((N, 1), jnp.float32),
      ],
  )(zp2, y2, dis, b2.reshape(1, H2), Wout, bout.reshape(1, 1))

  return (logits.squeeze(-1), embedding)
